# baseline (speedup 1.0000x reference)
import functools

import jax
import jax.numpy as jnp
from jax import lax
from jax.experimental import pallas as pl
from jax.experimental.pallas import tpu as pltpu
from jax.experimental.pallas import tpu_sc as plsc

_NC = 2
_NS = 16
_L = 16

_PREC = None


@functools.partial(jax.jit, static_argnames=("half_pad", "e_pad", "hid",
                                             "idx_blk", "chunk", "sub"))
def _seg_sum(x, src, dst, *, half_pad, e_pad, hid, idx_blk, chunk, sub):
    n_sub = chunk // sub
    per_tile = e_pad // _NS
    n_chunks = per_tile // chunk
    ch_per_blk = idx_blk // chunk
    n_blk = per_tile // idx_blk
    assert n_blk % 2 == 0 and ch_per_blk == 4
    rows_t = half_pad // _NS

    mesh = plsc.VectorSubcoreMesh(core_axis_name="c", subcore_axis_name="s")

    @functools.partial(
        pl.kernel,
        out_type=jax.ShapeDtypeStruct((2 * half_pad, hid), jnp.float32),
        mesh=mesh,
        compiler_params=pltpu.CompilerParams(use_tc_tiling_on_sc=False),
        scratch_types=[
            pltpu.VMEM((2, idx_blk), jnp.int32),
            pltpu.VMEM((2, idx_blk), jnp.int32),
            pltpu.VMEM((2, n_sub, sub), jnp.int32),
            pltpu.VMEM((2, chunk, hid), jnp.float32),
            pltpu.VMEM_SHARED((half_pad + 8, hid), jnp.float32),
            pltpu.SemaphoreType.DMA,
            (pltpu.SemaphoreType.DMA, pltpu.SemaphoreType.DMA),
            (pltpu.SemaphoreType.DMA, pltpu.SemaphoreType.DMA),
        ],
    )
    def seg(x_hbm, src_hbm, dst_hbm, zrows_hbm, out_hbm,
            src_v, dst_v, lidx_v, rows_v, acc, isem, gsems, ssems):
        c = lax.axis_index("c")
        s = lax.axis_index("s")
        off = c * half_pad
        base0 = s * per_tile

        pltpu.sync_copy(zrows_hbm, acc.at[pl.ds(s * rows_t, rows_t)])
        plsc.subcore_barrier()

        def fire_gather(u, v, coff):
            return [
                pltpu.async_copy(
                    x_hbm.at[src_v.at[v, pl.ds(coff + j * sub, sub)]],
                    rows_v.at[u, pl.ds(j * sub, sub)],
                    gsems[u],
                )
                for j in range(n_sub)
            ]

        def wait_gather(u, v, coff):
            for j in range(n_sub):
                pltpu.make_async_copy(
                    x_hbm.at[src_v.at[v, pl.ds(coff + j * sub, sub)]],
                    rows_v.at[u, pl.ds(j * sub, sub)],
                    gsems[u],
                ).wait()

        def fire_scatter(u):
            return [
                pltpu.async_copy(
                    rows_v.at[u, pl.ds(j * sub, sub)],
                    acc.at[lidx_v.at[u, j]],
                    ssems[u],
                    add=True,
                )
                for j in range(n_sub)
            ]

        def wait_scatter(u):
            for j in range(n_sub):
                pltpu.make_async_copy(
                    rows_v.at[u, pl.ds(j * sub, sub)],
                    acc.at[lidx_v.at[u, j]],
                    ssems[u],
                ).wait()

        def remap(u, v, coff):
            per_row = sub // _L
            for k in range(chunk // _L):
                d = dst_v[v, pl.ds(coff + k * _L, _L)]
                l = d - off
                ok = (d >= off) & (l < half_pad)
                l = jnp.where(ok, l, half_pad)
                lidx_v[u, k // per_row, pl.ds((k % per_row) * _L, _L)] = l

        pltpu.sync_copy(src_hbm.at[pl.ds(base0, idx_blk)], src_v.at[0])
        pltpu.sync_copy(dst_hbm.at[pl.ds(base0, idx_blk)], dst_v.at[0])
        fire_gather(0, 0, 0)

        def body(k2, carry):
            for j in range(2 * ch_per_blk):
                u = j % 2
                v = (j // 4) % 2
                vn = ((j + 1) // 4) % 2
                coff = (j % 4) * chunk
                coffn = ((j + 1) % 4) * chunk
                if j in (0, 4):
                    nb = 2 * k2 + j // 4 + 1
                    bb = jnp.minimum(nb, n_blk - 1)
                    base = base0 + bb * idx_blk
                    pltpu.async_copy(src_hbm.at[pl.ds(base, idx_blk)],
                                     src_v.at[nb % 2], isem)
                    pltpu.async_copy(dst_hbm.at[pl.ds(base, idx_blk)],
                                     dst_v.at[nb % 2], isem)
                remap(u, v, coff)
                if j == 0:
                    @pl.when(k2 > 0)
                    def _():
                        wait_scatter(1 - u)
                else:
                    wait_scatter(1 - u)
                if j in (3, 7):
                    nb = 2 * k2 + j // 4 + 1
                    pltpu.make_async_copy(
                        src_hbm.at[pl.ds(base0, idx_blk)],
                        src_v.at[nb % 2], isem).wait()
                    pltpu.make_async_copy(
                        dst_hbm.at[pl.ds(base0, idx_blk)],
                        dst_v.at[nb % 2], isem).wait()
                fire_gather(1 - u, vn, coffn)
                wait_gather(u, v, coff)
                fire_scatter(u)
            return carry

        lax.fori_loop(0, n_blk // 2, body, 0)
        wait_scatter((n_chunks - 1) % 2)
        wait_gather(n_chunks % 2, 0, 0)
        plsc.subcore_barrier()

        r0 = s * rows_t
        pltpu.sync_copy(acc.at[pl.ds(r0, rows_t)],
                        out_hbm.at[pl.ds(off + r0, rows_t)])

    zrows = jnp.zeros((rows_t, hid), jnp.float32)
    return seg(x, src, dst, zrows)


def _lookup_body(ops_ref, emb_ref, wm_ref, x_ref, y_ref):
    ops = ops_ref[0, 0, :]
    p = emb_ref.shape[0]
    onehot = (ops[:, None] == lax.broadcasted_iota(jnp.int32, (ops.shape[0], p), 1)
              ).astype(jnp.float32)
    x = jnp.dot(onehot, emb_ref[...], precision=lax.Precision.HIGHEST)
    x_ref[...] = x
    y_ref[...] = jnp.dot(x, wm_ref[...], precision=_PREC)


def _gru_body(x_ref, s_ref, wz_ref, uz_ref, bz_ref, wr_ref, ur_ref,
              br_ref, wh_ref, uh_ref, bh_ref, wm_ref, o_ref, y_ref):
    x = x_ref[...]
    m = s_ref[...]
    z = jax.nn.sigmoid(jnp.dot(m, wz_ref[...], precision=_PREC)
                       + jnp.dot(x, uz_ref[...], precision=_PREC) + bz_ref[...])
    r = jax.nn.sigmoid(jnp.dot(m, wr_ref[...], precision=_PREC)
                       + jnp.dot(x, ur_ref[...], precision=_PREC) + br_ref[...])
    h = jnp.tanh(jnp.dot(m, wh_ref[...], precision=_PREC)
                 + jnp.dot(r * x, uh_ref[...], precision=_PREC) + bh_ref[...])
    xn = (1.0 - z) * x + z * h
    o_ref[...] = xn
    y_ref[...] = jnp.dot(xn, wm_ref[...], precision=_PREC)


def _dec_body(x_ref, w1_ref, b1_ref, w2_ref, b2_ref, o_ref):
    a = jax.nn.relu(jnp.dot(x_ref[...], w1_ref[...], precision=_PREC)
                    + b1_ref[...])
    o_ref[...] = jnp.dot(a, w2_ref[...], precision=_PREC) + b2_ref[...]


def _full(shape):
    return pl.BlockSpec(shape, lambda i: (0,) * len(shape))


def _rows(bn, d):
    return pl.BlockSpec((bn, d), lambda i: (i, 0))


def kernel(node_ops, edge_index, embed, W_msg, W_z, U_z, b_z, W_r, U_r, b_r,
           W_h, U_h, b_h, W1, b1, W2, b2):
    n = node_ops.shape[0]
    e = edge_index.shape[1]
    hid = embed.shape[1]
    t_steps = 3

    idx_blk, chunk, sub = 1024, 256, 128
    n_blk = ((e // _NS) + idx_blk - 1) // idx_blk
    if n_blk % 2:
        n_blk += 1
    per_tile_pad = n_blk * idx_blk
    e_pad = per_tile_pad * _NS
    half_pad = (((n + 1) // 2) + 127) // 128 * 128

    src = edge_index[0].astype(jnp.int32)
    dst = edge_index[1].astype(jnp.int32)
    if e_pad > e:
        src = jnp.concatenate([src, jnp.zeros((e_pad - e,), jnp.int32)])
        dst = jnp.concatenate([dst, jnp.full((e_pad - e,), n, jnp.int32)])

    for bn in (5000, 2500, 2000, 1000, 500, 200, 104, 8, n):
        if n % bn == 0:
            break
    g = n // bn

    p_pad = 16
    emb_p = jnp.pad(embed, ((0, p_pad - embed.shape[0]), (0, 0)))
    ops3 = node_ops.astype(jnp.int32).reshape(g, 1, bn)
    x, y = pl.pallas_call(
        _lookup_body,
        grid=(g,),
        in_specs=[pl.BlockSpec((1, 1, bn), lambda i: (i, 0, 0)),
                  _full((p_pad, hid)), _full((hid, hid))],
        out_specs=[_rows(bn, hid), _rows(bn, hid)],
        out_shape=[jax.ShapeDtypeStruct((n, hid), jnp.float32),
                   jax.ShapeDtypeStruct((n, hid), jnp.float32)],
    )(ops3, emb_p, W_msg)

    b_z2, b_r2, b_h2 = (b.reshape(1, hid) for b in (b_z, b_r, b_h))
    gru = pl.pallas_call(
        _gru_body,
        grid=(g,),
        in_specs=[_rows(bn, hid), _rows(bn, hid)]
        + [_full((hid, hid)), _full((hid, hid)), _full((1, hid)),
           _full((hid, hid)), _full((hid, hid)), _full((1, hid)),
           _full((hid, hid)), _full((hid, hid)), _full((1, hid)),
           _full((hid, hid))],
        out_specs=[_rows(bn, hid), _rows(bn, hid)],
        out_shape=[jax.ShapeDtypeStruct((n, hid), jnp.float32),
                   jax.ShapeDtypeStruct((n, hid), jnp.float32)],
    )

    for _ in range(t_steps):
        s = _seg_sum(y, src, dst, half_pad=half_pad, e_pad=e_pad, hid=hid,
                     idx_blk=idx_blk, chunk=chunk, sub=sub)
        x, y = gru(x, s, W_z, U_z, b_z2, W_r, U_r, b_r2,
                   W_h, U_h, b_h2, W_msg)

    d2 = W2.shape[1]
    out = pl.pallas_call(
        _dec_body,
        grid=(g,),
        in_specs=[_rows(bn, hid), _full((hid, 2 * hid)),
                  _full((1, 2 * hid)), _full((2 * hid, d2)),
                  _full((1, d2))],
        out_specs=_rows(bn, d2),
        out_shape=jax.ShapeDtypeStruct((n, d2), jnp.float32),
    )(x, W1, b1.reshape(1, 2 * hid), W2, b2.reshape(1, d2))
    return out


# speedup vs baseline: 1.0136x; 1.0136x over previous
import functools

import jax
import jax.numpy as jnp
from jax import lax
from jax.experimental import pallas as pl
from jax.experimental.pallas import tpu as pltpu
from jax.experimental.pallas import tpu_sc as plsc

_NC = 2
_NS = 16
_L = 16

_PREC = None


@functools.partial(jax.jit, static_argnames=("half_pad", "e_pad", "hid", "chunk", "sub"))
def _seg_sum(x, src, dst, *, half_pad, e_pad, hid, chunk, sub):
    n_sub = chunk // sub
    per_tile = e_pad // _NS
    n_chunks = per_tile // chunk
    rows_t = half_pad // _NS

    mesh = plsc.VectorSubcoreMesh(core_axis_name="c", subcore_axis_name="s")

    @functools.partial(
        pl.kernel,
        out_type=jax.ShapeDtypeStruct((2 * half_pad, hid), jnp.float32),
        mesh=mesh,
        compiler_params=pltpu.CompilerParams(use_tc_tiling_on_sc=False,
                                             internal_scratch_in_bytes=0),
        scratch_types=[
            pltpu.VMEM((chunk,), jnp.int32),
            pltpu.VMEM((chunk,), jnp.int32),
            pltpu.VMEM((n_sub, sub), jnp.int32),
            pltpu.VMEM((chunk, hid), jnp.float32),
            pltpu.VMEM_SHARED((half_pad + 8, hid), jnp.float32),
            pltpu.SemaphoreType.DMA,
            pltpu.SemaphoreType.DMA,
        ],
    )
    def seg(x_hbm, src_hbm, dst_hbm, zrows_hbm, out_hbm,
            src_v, dst_v, lidx_v, rows_v, acc, gsem, ssem):
        c = lax.axis_index("c")
        s = lax.axis_index("s")
        off = c * half_pad

        pltpu.sync_copy(zrows_hbm, acc.at[pl.ds(s * rows_t, rows_t)])
        plsc.subcore_barrier()

        base0 = s * per_tile

        def chunk_body(g, carry):
            base = base0 + g * chunk
            pltpu.sync_copy(src_hbm.at[pl.ds(base, chunk)], src_v)
            pltpu.sync_copy(dst_hbm.at[pl.ds(base, chunk)], dst_v)
            gathers = [
                pltpu.async_copy(
                    x_hbm.at[src_v.at[pl.ds(j * sub, sub)]],
                    rows_v.at[pl.ds(j * sub, sub)],
                    gsem,
                )
                for j in range(n_sub)
            ]
            per_row = sub // _L
            for k in range(chunk // _L):
                d = dst_v[pl.ds(k * _L, _L)]
                l = d - off
                ok = (d >= off) & (l < half_pad)
                l = jnp.where(ok, l, half_pad)
                lidx_v[k // per_row, pl.ds((k % per_row) * _L, _L)] = l
            for cp in gathers:
                cp.wait()
            scatters = [
                pltpu.async_copy(
                    rows_v.at[pl.ds(j * sub, sub)],
                    acc.at[lidx_v.at[j]],
                    ssem,
                    add=True,
                )
                for j in range(n_sub)
            ]
            for cp in scatters:
                cp.wait()
            return carry

        lax.fori_loop(0, n_chunks, chunk_body, 0)
        plsc.subcore_barrier()

        r0 = s * rows_t
        pltpu.sync_copy(acc.at[pl.ds(r0, rows_t)],
                        out_hbm.at[pl.ds(off + r0, rows_t)])

    zrows = jnp.zeros((rows_t, hid), jnp.float32)
    return seg(x, src, dst, zrows)


def _lookup_body(ops_ref, emb_ref, wm_ref, x_ref, y_ref):
    ops = ops_ref[0, 0, :]
    p = emb_ref.shape[0]
    onehot = (ops[:, None] == lax.broadcasted_iota(jnp.int32, (ops.shape[0], p), 1)
              ).astype(jnp.float32)
    x = jnp.dot(onehot, emb_ref[...], precision=lax.Precision.HIGHEST)
    x_ref[...] = x
    y_ref[...] = jnp.dot(x, wm_ref[...], precision=_PREC)


def _gru_body(x_ref, s_ref, wz_ref, uz_ref, bz_ref, wr_ref, ur_ref,
              br_ref, wh_ref, uh_ref, bh_ref, wm_ref, o_ref, y_ref):
    x = x_ref[...]
    m = s_ref[...]
    z = jax.nn.sigmoid(jnp.dot(m, wz_ref[...], precision=_PREC)
                       + jnp.dot(x, uz_ref[...], precision=_PREC) + bz_ref[...])
    r = jax.nn.sigmoid(jnp.dot(m, wr_ref[...], precision=_PREC)
                       + jnp.dot(x, ur_ref[...], precision=_PREC) + br_ref[...])
    h = jnp.tanh(jnp.dot(m, wh_ref[...], precision=_PREC)
                 + jnp.dot(r * x, uh_ref[...], precision=_PREC) + bh_ref[...])
    xn = (1.0 - z) * x + z * h
    o_ref[...] = xn
    y_ref[...] = jnp.dot(xn, wm_ref[...], precision=_PREC)


def _dec_body(x_ref, w1_ref, b1_ref, w2_ref, b2_ref, o_ref):
    a = jax.nn.relu(jnp.dot(x_ref[...], w1_ref[...], precision=_PREC)
                    + b1_ref[...])
    o_ref[...] = jnp.dot(a, w2_ref[...], precision=_PREC) + b2_ref[...]


def _full(shape):
    return pl.BlockSpec(shape, lambda i: (0,) * len(shape))


def _rows(bn, d):
    return pl.BlockSpec((bn, d), lambda i: (i, 0))


def kernel(node_ops, edge_index, embed, W_msg, W_z, U_z, b_z, W_r, U_r, b_r,
           W_h, U_h, b_h, W1, b1, W2, b2):
    n = node_ops.shape[0]
    e = edge_index.shape[1]
    hid = embed.shape[1]
    t_steps = 3

    chunk, sub = 512, 128
    per_tile_pad = ((e // _NS) + chunk - 1) // chunk * chunk
    e_pad = per_tile_pad * _NS
    half_pad = (((n + 1) // 2) + 127) // 128 * 128

    src = edge_index[0].astype(jnp.int32)
    dst = edge_index[1].astype(jnp.int32)
    if e_pad > e:
        src = jnp.concatenate([src, jnp.zeros((e_pad - e,), jnp.int32)])
        dst = jnp.concatenate([dst, jnp.full((e_pad - e,), n, jnp.int32)])

    for bn in (5000, 2500, 2000, 1000, 500, 200, 104, 8, n):
        if n % bn == 0:
            break
    g = n // bn

    p_pad = 16
    emb_p = jnp.pad(embed, ((0, p_pad - embed.shape[0]), (0, 0)))
    ops3 = node_ops.astype(jnp.int32).reshape(g, 1, bn)
    x, y = pl.pallas_call(
        _lookup_body,
        grid=(g,),
        in_specs=[pl.BlockSpec((1, 1, bn), lambda i: (i, 0, 0)),
                  _full((p_pad, hid)), _full((hid, hid))],
        out_specs=[_rows(bn, hid), _rows(bn, hid)],
        out_shape=[jax.ShapeDtypeStruct((n, hid), jnp.float32),
                   jax.ShapeDtypeStruct((n, hid), jnp.float32)],
    )(ops3, emb_p, W_msg)

    b_z2, b_r2, b_h2 = (b.reshape(1, hid) for b in (b_z, b_r, b_h))
    gru = pl.pallas_call(
        _gru_body,
        grid=(g,),
        in_specs=[_rows(bn, hid), _rows(bn, hid)]
        + [_full((hid, hid)), _full((hid, hid)), _full((1, hid)),
           _full((hid, hid)), _full((hid, hid)), _full((1, hid)),
           _full((hid, hid)), _full((hid, hid)), _full((1, hid)),
           _full((hid, hid))],
        out_specs=[_rows(bn, hid), _rows(bn, hid)],
        out_shape=[jax.ShapeDtypeStruct((n, hid), jnp.float32),
                   jax.ShapeDtypeStruct((n, hid), jnp.float32)],
    )

    for _ in range(t_steps):
        s = _seg_sum(y, src, dst, half_pad=half_pad, e_pad=e_pad, hid=hid,
                     chunk=chunk, sub=sub)
        x, y = gru(x, s, W_z, U_z, b_z2, W_r, U_r, b_r2,
                   W_h, U_h, b_h2, W_msg)

    d2 = W2.shape[1]
    out = pl.pallas_call(
        _dec_body,
        grid=(g,),
        in_specs=[_rows(bn, hid), _full((hid, 2 * hid)),
                  _full((1, 2 * hid)), _full((2 * hid, d2)),
                  _full((1, d2))],
        out_specs=_rows(bn, d2),
        out_shape=jax.ShapeDtypeStruct((n, d2), jnp.float32),
    )(x, W1, b1.reshape(1, 2 * hid), W2, b2.reshape(1, d2))
    return out
